# baseline (speedup 1.0000x reference)
import jax
import jax.numpy as jnp
from jax.experimental import pallas as pl
from jax.experimental.pallas import tpu as pltpu

RANK = 16
PADK = 128
TBLK = 512


def _lra_block(x_ref, ri_ref, colkey_ref, wd_ref, wu_ref, o_ref):
    xb = x_ref[...]
    ri = ri_ref[...]
    mask = colkey_ref[...] == ri
    down = jnp.dot(xb.astype(jnp.bfloat16), wd_ref[...],
                   preferred_element_type=jnp.float32)
    down = jnp.where(mask, down, 0.0)
    delta = jnp.dot(down.astype(jnp.bfloat16), wu_ref[...],
                    preferred_element_type=jnp.float32)
    o_ref[...] = xb + delta


def kernel(x, router_indices, LRA_mask, W_down, W_up):
    ntok, dim = x.shape
    nkeys, rank, _ = W_down.shape

    wd = jnp.transpose(W_down[LRA_mask], (2, 0, 1)).reshape(dim, nkeys * rank)
    wd = jnp.pad(wd, ((0, 0), (0, PADK - nkeys * rank))).astype(jnp.bfloat16)
    wu = jnp.transpose(W_up[LRA_mask], (0, 2, 1)).reshape(nkeys * rank, dim)
    wu = jnp.pad(wu, ((0, PADK - nkeys * rank), (0, 0))).astype(jnp.bfloat16)
    colkey = jnp.pad(jnp.repeat(LRA_mask, rank), (0, PADK - nkeys * rank),
                     constant_values=-1).reshape(1, PADK)

    grid = (ntok // TBLK,)
    return pl.pallas_call(
        _lra_block,
        grid=grid,
        in_specs=[
            pl.BlockSpec((TBLK, dim), lambda i: (i, 0)),
            pl.BlockSpec((TBLK, 1), lambda i: (i, 0)),
            pl.BlockSpec((1, PADK), lambda i: (0, 0)),
            pl.BlockSpec((dim, PADK), lambda i: (0, 0)),
            pl.BlockSpec((PADK, dim), lambda i: (0, 0)),
        ],
        out_specs=pl.BlockSpec((TBLK, dim), lambda i: (i, 0)),
        out_shape=jax.ShapeDtypeStruct((ntok, dim), x.dtype),
        compiler_params=pltpu.CompilerParams(
            dimension_semantics=("arbitrary",),
        ),
    )(x, router_indices, colkey, wd, wu)


# speedup vs baseline: 1.1536x; 1.1536x over previous
import jax
import jax.numpy as jnp
from jax.experimental import pallas as pl
from jax.experimental.pallas import tpu as pltpu

RANK = 16
PADK = 128
TBLK = 1024
NCHUNK = 4
CHUNK = TBLK // NCHUNK


def _lra_block(x_ref, ri_ref, colkey_ref, wd_ref, wu_ref, o_ref):
    colkey = colkey_ref[...]
    for h in range(NCHUNK):
        lo, hi = h * CHUNK, (h + 1) * CHUNK
        xb = x_ref[lo:hi, :]
        ri = ri_ref[lo:hi, :]
        mask = colkey == ri
        down = jnp.dot(xb.astype(jnp.bfloat16), wd_ref[...],
                       preferred_element_type=jnp.float32)
        down = jnp.where(mask, down.astype(jnp.bfloat16), jnp.bfloat16(0))
        delta = jnp.dot(down, wu_ref[...],
                        preferred_element_type=jnp.float32)
        o_ref[lo:hi, :] = xb + delta


def kernel(x, router_indices, LRA_mask, W_down, W_up):
    ntok, dim = x.shape
    nkeys, rank, _ = W_down.shape

    wd = jnp.transpose(W_down[LRA_mask], (2, 0, 1)).reshape(dim, nkeys * rank)
    wd = jnp.pad(wd, ((0, 0), (0, PADK - nkeys * rank))).astype(jnp.bfloat16)
    wu = jnp.transpose(W_up[LRA_mask], (0, 2, 1)).reshape(nkeys * rank, dim)
    wu = jnp.pad(wu, ((0, PADK - nkeys * rank), (0, 0))).astype(jnp.bfloat16)
    colkey = jnp.pad(jnp.repeat(LRA_mask, rank), (0, PADK - nkeys * rank),
                     constant_values=-1).reshape(1, PADK)

    grid = (ntok // TBLK,)
    return pl.pallas_call(
        _lra_block,
        grid=grid,
        in_specs=[
            pl.BlockSpec((TBLK, dim), lambda i: (i, 0)),
            pl.BlockSpec((TBLK, 1), lambda i: (i, 0)),
            pl.BlockSpec((1, PADK), lambda i: (0, 0)),
            pl.BlockSpec((dim, PADK), lambda i: (0, 0)),
            pl.BlockSpec((PADK, dim), lambda i: (0, 0)),
        ],
        out_specs=pl.BlockSpec((TBLK, dim), lambda i: (i, 0)),
        out_shape=jax.ShapeDtypeStruct((ntok, dim), x.dtype),
        compiler_params=pltpu.CompilerParams(
            dimension_semantics=("arbitrary",),
        ),
    )(x, router_indices, colkey, wd, wu)


# speedup vs baseline: 1.1926x; 1.0338x over previous
import jax
import jax.numpy as jnp
from jax.experimental import pallas as pl
from jax.experimental.pallas import tpu as pltpu

RANK = 16
PADK = 128
TILE = 512
NBUF = 6


def _lra_pipe(x_hbm, ri_ref, colkey_ref, wd_ref, wu_ref, o_hbm,
              xbuf, obuf, insem, outsem):
    ntok = x_hbm.shape[0]
    ntiles = ntok // TILE
    colkey = colkey_ref[...]

    def in_copy(t, slot):
        return pltpu.make_async_copy(
            x_hbm.at[pl.ds(t * TILE, TILE), :], xbuf.at[slot], insem.at[slot])

    def out_copy(t, slot):
        return pltpu.make_async_copy(
            obuf.at[slot], o_hbm.at[pl.ds(t * TILE, TILE), :], outsem.at[slot])

    for k in range(NBUF):
        in_copy(k, k).start()

    def step(i, carry):
        slot = jax.lax.rem(i, NBUF)

        @pl.when(i >= NBUF)
        def _():
            out_copy(i - NBUF, slot).wait()

        in_copy(i, slot).wait()
        xb = xbuf[slot]
        ri = ri_ref[pl.ds(i * TILE, TILE), :]
        mask = colkey == ri
        down = jnp.dot(xb.astype(jnp.bfloat16), wd_ref[...],
                       preferred_element_type=jnp.float32)
        down = jnp.where(mask, down.astype(jnp.bfloat16), jnp.bfloat16(0))
        delta = jnp.dot(down, wu_ref[...],
                        preferred_element_type=jnp.float32)
        obuf[slot] = xb + delta

        @pl.when(i + NBUF < ntiles)
        def _():
            in_copy(i + NBUF, slot).start()

        out_copy(i, slot).start()
        return carry

    jax.lax.fori_loop(0, ntiles, step, 0)
    for k in range(NBUF):
        t = ntiles - NBUF + k
        out_copy(t, t % NBUF).wait()


def kernel(x, router_indices, LRA_mask, W_down, W_up):
    ntok, dim = x.shape
    nkeys, rank, _ = W_down.shape

    wd = jnp.transpose(W_down[LRA_mask], (2, 0, 1)).reshape(dim, nkeys * rank)
    wd = jnp.pad(wd, ((0, 0), (0, PADK - nkeys * rank))).astype(jnp.bfloat16)
    wu = jnp.transpose(W_up[LRA_mask], (0, 2, 1)).reshape(nkeys * rank, dim)
    wu = jnp.pad(wu, ((0, PADK - nkeys * rank), (0, 0))).astype(jnp.bfloat16)
    colkey = jnp.pad(jnp.repeat(LRA_mask, rank), (0, PADK - nkeys * rank),
                     constant_values=-1).reshape(1, PADK)

    return pl.pallas_call(
        _lra_pipe,
        in_specs=[
            pl.BlockSpec(memory_space=pl.ANY),
            pl.BlockSpec(memory_space=pltpu.VMEM),
            pl.BlockSpec(memory_space=pltpu.VMEM),
            pl.BlockSpec(memory_space=pltpu.VMEM),
            pl.BlockSpec(memory_space=pltpu.VMEM),
        ],
        out_specs=pl.BlockSpec(memory_space=pl.ANY),
        out_shape=jax.ShapeDtypeStruct((ntok, dim), x.dtype),
        scratch_shapes=[
            pltpu.VMEM((NBUF, TILE, dim), jnp.float32),
            pltpu.VMEM((NBUF, TILE, dim), jnp.float32),
            pltpu.SemaphoreType.DMA((NBUF,)),
            pltpu.SemaphoreType.DMA((NBUF,)),
        ],
    )(x, router_indices, colkey, wd, wu)
